# Initial kernel scaffold; baseline (speedup 1.0000x reference)
import jax
import jax.numpy as jnp
from jax.experimental import pallas as pl


def kernel(X, tables, W1, b1, W2, b2, W3, b3):
    raise NotImplementedError("write your pallas kernel here")


# speedup vs baseline: 7.6555x; 7.6555x over previous
import functools

import jax
import jax.numpy as jnp
from jax import lax
from jax.experimental import pallas as pl
from jax.experimental.pallas import tpu as pltpu
from jax.experimental.pallas import tpu_sc as plsc

B = 16384
F = 26
V = 100000
D = 16

_INFO = plsc.get_sparse_core_info()
NC = _INFO.num_cores
NS = _INFO.num_subcores
LN = _INFO.num_lanes
NW = NC * NS

PER_W = (B * F) // NW
NROW = 128
NG = PER_W // NROW
GB = 13
NGROUP = NG // GB


def _sc_gather_body(x_hbm, offs_hbm, tab_hbm, out_hbm, idx_v, offs_v, rows_v, gsem):
    wid = lax.axis_index("s") * NC + lax.axis_index("c")
    base = wid * PER_W

    pltpu.sync_copy(x_hbm.at[wid], idx_v)
    pltpu.sync_copy(offs_hbm, offs_v)

    def add_body(i, carry):
        g = i // (NROW // LN)
        c = (i % (NROW // LN)) * LN
        sl = pl.ds(c, LN)
        idx_v[g, sl] = idx_v[g, sl] + offs_v[g, sl]
        return carry

    lax.fori_loop(0, NG * (NROW // LN), add_body, 0)

    def group_body(gi, carry):
        cps = [
            pltpu.make_async_copy(
                tab_hbm.at[idx_v.at[gi * GB + k]],
                rows_v.at[pl.ds(k * NROW, NROW)],
                gsem,
            )
            for k in range(GB)
        ]
        for cp in cps:
            cp.start()
        for cp in cps:
            cp.wait()
        pltpu.sync_copy(rows_v, out_hbm.at[pl.ds(base + gi * (GB * NROW), GB * NROW)])
        return carry

    lax.fori_loop(0, NGROUP, group_body, 0)


@functools.partial(jax.jit, static_argnames=())
def _sc_gather(x3d, offs2d, tab2d):
    return pl.kernel(
        _sc_gather_body,
        out_type=jax.ShapeDtypeStruct((B * F, D), jnp.float32),
        mesh=plsc.VectorSubcoreMesh(core_axis_name="c", subcore_axis_name="s"),
        scratch_types=[
            pltpu.VMEM((NG, NROW), jnp.int32),
            pltpu.VMEM((NG, NROW), jnp.int32),
            pltpu.VMEM((GB * NROW, D), jnp.float32),
            pltpu.SemaphoreType.DMA,
        ],
        compiler_params=pltpu.CompilerParams(use_tc_tiling_on_sc=False),
    )(x3d, offs2d, tab2d)


BT = 2048


def _mlp_body(emb_ref, w1_ref, b1_ref, w2_ref, b2_ref, w3_ref, b3_ref, out_ref):
    h = jnp.dot(emb_ref[...], w1_ref[...], preferred_element_type=jnp.float32)
    h = jnp.maximum(h + b1_ref[...], 0.0)
    h = jnp.dot(h, w2_ref[...], preferred_element_type=jnp.float32)
    h = jnp.maximum(h + b2_ref[...], 0.0)
    o = jnp.dot(h, w3_ref[...], preferred_element_type=jnp.float32) + b3_ref[...]
    out_ref[...] = 1.0 / (1.0 + jnp.exp(-o))


def _mlp(emb, W1, b1, W2, b2, W3, b3):
    FD = F * D
    return pl.pallas_call(
        _mlp_body,
        grid=(B // BT,),
        in_specs=[
            pl.BlockSpec((BT, FD), lambda i: (i, 0)),
            pl.BlockSpec((FD, 1000), lambda i: (0, 0)),
            pl.BlockSpec((1, 1000), lambda i: (0, 0)),
            pl.BlockSpec((1000, 500), lambda i: (0, 0)),
            pl.BlockSpec((1, 500), lambda i: (0, 0)),
            pl.BlockSpec((500, 1), lambda i: (0, 0)),
            pl.BlockSpec((1, 1), lambda i: (0, 0)),
        ],
        out_specs=pl.BlockSpec((BT, 1), lambda i: (i, 0)),
        out_shape=jax.ShapeDtypeStruct((B, 1), jnp.float32),
        compiler_params=pltpu.CompilerParams(
            dimension_semantics=("arbitrary",),
        ),
    )(emb, W1, b1, W2, b2, W3, b3)


def kernel(X, tables, W1, b1, W2, b2, W3, b3):
    tab2d = tables.reshape(F * V, D)
    x3d = X.reshape(NW, NG, NROW)
    offs2d = ((jnp.arange(PER_W, dtype=jnp.int32) % F) * V).reshape(NG, NROW)
    emb2d = _sc_gather(x3d, offs2d, tab2d)
    emb = emb2d.reshape(B, F * D)
    return _mlp(
        emb,
        W1, b1.reshape(1, 1000),
        W2, b2.reshape(1, 500),
        W3, b3.reshape(1, 1),
    )
